# Initial kernel scaffold; baseline (speedup 1.0000x reference)
import jax
import jax.numpy as jnp
from jax.experimental import pallas as pl


def kernel(data, grid, probe_geometry, t0_delays, initial_times):
    raise NotImplementedError("write your pallas kernel here")


# speedup vs baseline: 1831.9436x; 1831.9436x over previous
import functools

import jax
import jax.numpy as jnp
from jax import lax
from jax.experimental import pallas as pl
from jax.experimental.pallas import tpu as pltpu
from jax.experimental.pallas import tpu_sc as plsc

N_TX = 4
N_EL = 128
N_AX = 2048
N_CH = 2
NZ = 256
NX = 128
NPIX = NZ * NX
FS = 25e6
FC = 6.25e6
C = 1540.0
F_NUMBER = 1.0

NC = 2
NS = 16
NW = NC * NS
PPW = NPIX // NW
L = 16
NG = PPW // L

_BP = 2048
_NBLK = NPIX // _BP


def _pre_body(gx_ref, gy_ref, gz_ref, p_ref, t0c_ref, itfs_ref,
              rx_ref, cb_ref, sb_ref, txd_ref, ca_ref, sa_ref):
    gx = gx_ref[0, 0, :]
    gy = gy_ref[0, 0, :]
    gz = gz_ref[0, 0, :]
    px = p_ref[0, :][:, None]
    py = p_ref[1, :][:, None]
    pz = p_ref[2, :][:, None]

    dx = gx[None, :] - px
    dy = gy[None, :] - py
    dz = gz[None, :] - pz
    dist = jnp.sqrt(dx * dx + dy * dy + dz * dz)
    rx = dist * (FS / C)
    rx_ref[:, :] = rx

    mask = (jnp.abs(dx) <= gz[None, :] * (0.5 / F_NUMBER)).astype(jnp.float32)
    b = rx * (jnp.pi / 2)
    cb_ref[:, :] = jnp.cos(b) * mask
    sb_ref[:, :] = jnp.sin(b) * mask

    t0c = t0c_ref[:, :]
    itfs = itfs_ref[:, 0][:, None]
    rows = []
    for t in range(N_TX):
        rows.append(jnp.min(t0c[t, :][:, None] + dist, axis=0))
    txdist = jnp.stack(rows, axis=0)
    txdel = txdist * (FS / C) - itfs
    txd_ref[:, :] = txdel
    a = txdel * (jnp.pi / 2) - gz[None, :] * (4 * jnp.pi * FC / C)
    ca_ref[:, :] = jnp.cos(a)
    sa_ref[:, :] = jnp.sin(a)


def _precompute(gx, gy, gz, p, t0c, itfs):
    f32 = jnp.float32
    return pl.pallas_call(
        _pre_body,
        grid=(_NBLK,),
        in_specs=[
            pl.BlockSpec((1, 1, _BP), lambda i: (i, 0, 0)),
            pl.BlockSpec((1, 1, _BP), lambda i: (i, 0, 0)),
            pl.BlockSpec((1, 1, _BP), lambda i: (i, 0, 0)),
            pl.BlockSpec((3, N_EL), lambda i: (0, 0)),
            pl.BlockSpec((N_TX, N_EL), lambda i: (0, 0)),
            pl.BlockSpec((N_TX, N_EL), lambda i: (0, 0)),
        ],
        out_specs=[
            pl.BlockSpec((N_EL, _BP), lambda i: (0, i)),
            pl.BlockSpec((N_EL, _BP), lambda i: (0, i)),
            pl.BlockSpec((N_EL, _BP), lambda i: (0, i)),
            pl.BlockSpec((N_TX, _BP), lambda i: (0, i)),
            pl.BlockSpec((N_TX, _BP), lambda i: (0, i)),
            pl.BlockSpec((N_TX, _BP), lambda i: (0, i)),
        ],
        out_shape=[
            jax.ShapeDtypeStruct((N_EL, NPIX), f32),
            jax.ShapeDtypeStruct((N_EL, NPIX), f32),
            jax.ShapeDtypeStruct((N_EL, NPIX), f32),
            jax.ShapeDtypeStruct((N_TX, NPIX), f32),
            jax.ShapeDtypeStruct((N_TX, NPIX), f32),
            jax.ShapeDtypeStruct((N_TX, NPIX), f32),
        ],
    )(gx, gy, gz, p, t0c, itfs)


def _sc_body(data_h, rx_h, cb_h, sb_h, txd_h, ca_h, sa_h, out_h,
             dat_v0, dat_v1, rx_v, cb_v, sb_v, txd_v, ca_v, sa_v, au_v, av_v, out_v,
             sem_d0, sem_d1, sem_s0, sem_s1):
    dat_v = (dat_v0, dat_v1)
    cid = lax.axis_index("c")
    sid = lax.axis_index("s")
    wid = sid * NC + cid
    base = wid * PPW

    sem_d = (sem_d0, sem_d1)
    sem_s = (sem_s0, sem_s1)

    pltpu.sync_copy(txd_h.at[:, pl.ds(base, PPW)], txd_v)
    pltpu.sync_copy(ca_h.at[:, pl.ds(base, PPW)], ca_v)
    pltpu.sync_copy(sa_h.at[:, pl.ds(base, PPW)], sa_v)

    zero = jnp.zeros((L,), jnp.float32)

    def zbody(g, c):
        sl = pl.ds(g * L, L)
        for t in range(N_TX):
            au_v[t, sl] = zero
            av_v[t, sl] = zero
        return c
    lax.fori_loop(0, NG, zbody, 0)

    def start(el, slot):
        for t in range(N_TX):
            pltpu.async_copy(data_h.at[t, el],
                             dat_v[slot].at[pl.ds(t * N_AX * N_CH, N_AX * N_CH)],
                             sem_d[slot])
        pltpu.async_copy(rx_h.at[el, pl.ds(base, PPW)], rx_v.at[slot], sem_s[slot])
        pltpu.async_copy(cb_h.at[el, pl.ds(base, PPW)], cb_v.at[slot], sem_s[slot])
        pltpu.async_copy(sb_h.at[el, pl.ds(base, PPW)], sb_v.at[slot], sem_s[slot])

    def wait(el, slot):
        for t in range(N_TX):
            pltpu.make_async_copy(data_h.at[t, el],
                                  dat_v[slot].at[pl.ds(t * N_AX * N_CH, N_AX * N_CH)],
                                  sem_d[slot]).wait()
        pltpu.make_async_copy(rx_h.at[el, pl.ds(base, PPW)], rx_v.at[slot], sem_s[slot]).wait()
        pltpu.make_async_copy(cb_h.at[el, pl.ds(base, PPW)], cb_v.at[slot], sem_s[slot]).wait()
        pltpu.make_async_copy(sb_h.at[el, pl.ds(base, PPW)], sb_v.at[slot], sem_s[slot]).wait()

    def compute(slot):
        def gbody(g, c):
            sl = pl.ds(g * L, L)
            rx = rx_v[slot, sl]
            cb = cb_v[slot, sl]
            sb = sb_v[slot, sl]
            for t in range(N_TX):
                d = txd_v[t, sl] + rx
                d0i = jnp.minimum(d.astype(jnp.int32), N_AX - 1)
                d1i = jnp.minimum(d0i + 1, N_AX - 1)
                w0 = d1i.astype(jnp.float32) - d
                w1 = d - d0i.astype(jnp.float32)
                f0 = (d0i << 1) + (t * N_AX * N_CH)
                f1 = (d1i << 1) + (t * N_AX * N_CH)
                i0 = plsc.load_gather(dat_v[slot], [f0])
                q0 = plsc.load_gather(dat_v[slot], [f0 + 1])
                i1 = plsc.load_gather(dat_v[slot], [f1])
                q1 = plsc.load_gather(dat_v[slot], [f1 + 1])
                ii = w0 * i0 + w1 * i1
                qq = w0 * q0 + w1 * q1
                au_v[t, sl] += ii * cb - qq * sb
                av_v[t, sl] += qq * cb + ii * sb
            return c
        lax.fori_loop(0, NG, gbody, 0)

    start(0, 0)

    def pair(i2, c):
        el0 = 2 * i2
        start(el0 + 1, 1)
        wait(el0, 0)
        compute(0)

        @pl.when(el0 + 2 < N_EL)
        def _():
            start(el0 + 2, 0)

        wait(el0 + 1, 1)
        compute(1)
        return c
    lax.fori_loop(0, N_EL // 2, pair, 0)

    def fbody(g, c):
        sl = pl.ds(g * L, L)
        oi = jnp.zeros((L,), jnp.float32)
        oq = jnp.zeros((L,), jnp.float32)
        for t in range(N_TX):
            u = au_v[t, sl]
            v = av_v[t, sl]
            cav = ca_v[t, sl]
            sav = sa_v[t, sl]
            oi = oi + (u * cav - v * sav)
            oq = oq + (v * cav + u * sav)
        out_v[0, sl] = oi
        out_v[1, sl] = oq
        return c
    lax.fori_loop(0, NG, fbody, 0)

    pltpu.sync_copy(out_v, out_h.at[:, pl.ds(base, PPW)])


def _das_sc(data, rx, cb, sb, txd, ca, sa):
    f32 = jnp.float32
    mesh = plsc.VectorSubcoreMesh(core_axis_name="c", subcore_axis_name="s",
                                  num_cores=NC, num_subcores=NS)
    fn = pl.kernel(
        _sc_body,
        out_type=jax.ShapeDtypeStruct((N_CH, NPIX), f32),
        mesh=mesh,
        compiler_params=pltpu.CompilerParams(needs_layout_passes=False),
        scratch_types=[
            pltpu.VMEM((N_TX * N_AX * N_CH,), f32),
            pltpu.VMEM((N_TX * N_AX * N_CH,), f32),
            pltpu.VMEM((2, PPW), f32),
            pltpu.VMEM((2, PPW), f32),
            pltpu.VMEM((2, PPW), f32),
            pltpu.VMEM((N_TX, PPW), f32),
            pltpu.VMEM((N_TX, PPW), f32),
            pltpu.VMEM((N_TX, PPW), f32),
            pltpu.VMEM((N_TX, PPW), f32),
            pltpu.VMEM((N_TX, PPW), f32),
            pltpu.VMEM((N_CH, PPW), f32),
            pltpu.SemaphoreType.DMA,
            pltpu.SemaphoreType.DMA,
            pltpu.SemaphoreType.DMA,
            pltpu.SemaphoreType.DMA,
        ],
    )
    return fn(data, rx, cb, sb, txd, ca, sa)


def kernel(data, grid, probe_geometry, t0_delays, initial_times):
    gx = grid[:, 0].reshape(_NBLK, 1, _BP)
    gy = grid[:, 1].reshape(_NBLK, 1, _BP)
    gz = grid[:, 2].reshape(_NBLK, 1, _BP)
    p = probe_geometry.T
    t0c = t0_delays * C
    itfs = jnp.broadcast_to((initial_times * FS)[:, None], (N_TX, N_EL))

    rx, cb, sb, txd, ca, sa = _precompute(gx, gy, gz, p, t0c, itfs)
    data_flat = data.reshape(N_TX, N_EL, N_AX * N_CH)
    out = _das_sc(data_flat, rx, cb, sb, txd, ca, sa)
    return out.T.reshape(NZ, NX, N_CH)
